# baseline (speedup 1.0000x reference)
import jax
import jax.numpy as jnp
from jax import lax
from jax.experimental import pallas as pl
from jax.experimental.pallas import tpu as pltpu
from jax.experimental.pallas import tpu_sc as plsc

N_TOKENS = 32768
B_SEGS = 16
N_WORKERS = 16
CHUNK = N_TOKENS // N_WORKERS
LANES = 16
UNROLL = 4
PAD = UNROLL * LANES


def _merge(am, ai, bm, bi):
    better = (bm > am) | ((bm == am) & (bi < ai))
    return jnp.where(better, bm, am), jnp.where(better, bi, ai)


def _tree_merge(pairs):
    while len(pairs) > 1:
        nxt = []
        for i in range(0, len(pairs) - 1, 2):
            am, ai = pairs[i]
            bm, bi = pairs[i + 1]
            nxt.append(_merge(am, ai, bm, bi))
        if len(pairs) % 2:
            nxt.append(pairs[-1])
        pairs = nxt
    return pairs[0]


def _jagged_argmax_body(values_hbm, ps_hbm, out_hbm,
                        vals_v, ps_v, pub_v, mrg_v, ans_v,
                        cmtab_v, citab_v, sh_pub, vals_sem, ps_sem):
    cid = lax.axis_index("c")
    sid = lax.axis_index("s")

    @pl.when(cid == 0)
    def _():
        lo = sid * CHUNK
        vals_cp = pltpu.async_copy(values_hbm.at[pl.ds(lo, CHUNK)],
                                   vals_v.at[pl.ds(0, CHUNK)], vals_sem)
        ps_cp = pltpu.async_copy(ps_hbm, ps_v, ps_sem)
        ps_cp.wait()

        lane = lax.iota(jnp.int32, LANES)
        ninf = jnp.full((LANES,), -jnp.inf, jnp.float32)
        sent = jnp.full((LANES,), N_TOKENS, jnp.int32)

        ps_vec = ps_v[...]
        for k in range(B_SEGS):
            cmtab_v[pl.ds(k * LANES, LANES)] = ninf
            citab_v[pl.ds(k * LANES, LANES)] = sent
        vals_cp.wait()
        for k in range(B_SEGS):
            start = ps_vec[k - 1] if k > 0 else jnp.int32(0)
            end = ps_vec[k]
            a = jnp.maximum(start, lo)
            b = jnp.minimum(end, lo + CHUNK)

            @pl.when(a < b)
            def _(a=a, b=b, lo=lo, k=k):
                va = lax.shift_right_logical(a, 4)
                vb = lax.shift_right_logical(b + (LANES - 1), 4)

                @pl.loop(va, vb, init_carry=(ninf, sent), step=UNROLL)
                def scan(j, carry):
                    cm, ci = carry
                    for u in range(UNROLL):
                        off = (j + u) * LANES
                        v = vals_v[pl.ds(off - lo, LANES)]
                        pos = off + lane
                        valid = (pos >= a) & (pos < b)
                        vm = jnp.where(valid, v, ninf)
                        upd = vm > cm
                        cm = jnp.where(upd, vm, cm)
                        ci = jnp.where(upd, pos, ci)
                    return cm, ci

                cm, ci = scan
                cmtab_v[pl.ds(k * LANES, LANES)] = cm
                citab_v[pl.ds(k * LANES, LANES)] = ci

        cols = []
        for l in range(LANES):
            tm = plsc.load_gather(cmtab_v, [lane * LANES + l])
            ti = plsc.load_gather(citab_v, [lane * LANES + l])
            cols.append((tm, ti))
        pm, pi = _tree_merge(cols)

        pub_v[pl.ds(0, LANES)] = pm
        pub_v[pl.ds(LANES, LANES)] = plsc.bitcast(pi, jnp.float32)
        pltpu.sync_copy(pub_v, sh_pub.at[pl.ds(sid * 2 * LANES, 2 * LANES)])
        plsc.subcore_barrier()

        @pl.when(sid == 0)
        def _():
            pltpu.sync_copy(sh_pub, mrg_v)
            rows = []
            for w in range(N_WORKERS):
                wm = mrg_v[pl.ds(w * 2 * LANES, LANES)]
                wi = plsc.bitcast(mrg_v[pl.ds(w * 2 * LANES + LANES, LANES)],
                                  jnp.int32)
                rows.append((wm, wi))
            cm, ci = _tree_merge(rows)
            ans_v[...] = jnp.where(ci >= N_TOKENS, -1, ci)
            pltpu.sync_copy(ans_v, out_hbm)


@jax.jit
def kernel(values, prefix_sum):
    ps32 = prefix_sum.astype(jnp.int32)
    mesh = plsc.VectorSubcoreMesh(
        core_axis_name="c", subcore_axis_name="s", num_cores=1, num_subcores=16
    )
    out = pl.kernel(
        _jagged_argmax_body,
        out_type=jax.ShapeDtypeStruct((B_SEGS,), jnp.int32),
        mesh=mesh,
        compiler_params=pltpu.CompilerParams(needs_layout_passes=False),
        scratch_types=[
            pltpu.VMEM((CHUNK + PAD,), jnp.float32),
            pltpu.VMEM((B_SEGS,), jnp.int32),
            pltpu.VMEM((2 * LANES,), jnp.float32),
            pltpu.VMEM((N_WORKERS * 2 * LANES,), jnp.float32),
            pltpu.VMEM((B_SEGS,), jnp.int32),
            pltpu.VMEM((B_SEGS * LANES,), jnp.float32),
            pltpu.VMEM((B_SEGS * LANES,), jnp.int32),
            pltpu.VMEM_SHARED((N_WORKERS * 2 * LANES,), jnp.float32),
            pltpu.SemaphoreType.DMA,
            pltpu.SemaphoreType.DMA,
        ],
    )(values, ps32)
    return out.astype(prefix_sum.dtype)


# speedup vs baseline: 1.0153x; 1.0153x over previous
import jax
import jax.numpy as jnp
from jax import lax
from jax.experimental import pallas as pl
from jax.experimental.pallas import tpu as pltpu
from jax.experimental.pallas import tpu_sc as plsc

N_TOKENS = 32768
B_SEGS = 16
N_WORKERS = 16
CHUNK = N_TOKENS // N_WORKERS
LANES = 16
UNROLL = 4
PAD = UNROLL * LANES


def _merge(am, ai, bm, bi):
    better = (bm > am) | ((bm == am) & (bi < ai))
    return jnp.where(better, bm, am), jnp.where(better, bi, ai)


def _tree_merge(pairs):
    while len(pairs) > 1:
        nxt = []
        for i in range(0, len(pairs) - 1, 2):
            am, ai = pairs[i]
            bm, bi = pairs[i + 1]
            nxt.append(_merge(am, ai, bm, bi))
        if len(pairs) % 2:
            nxt.append(pairs[-1])
        pairs = nxt
    return pairs[0]


def _jagged_argmax_body(values_hbm, ps_hbm, out_hbm,
                        vals_v, ps_v, pub_v, mrg_v, ans_v,
                        cmtab_v, citab_v, sh_pub, vals_sem, ps_sem):
    cid = lax.axis_index("c")
    sid = lax.axis_index("s")

    @pl.when(cid == 0)
    def _():
        lo = sid * CHUNK
        vals_cp = pltpu.async_copy(values_hbm.at[pl.ds(lo, CHUNK)],
                                   vals_v.at[pl.ds(0, CHUNK)], vals_sem)
        ps_cp = pltpu.async_copy(ps_hbm, ps_v, ps_sem)
        ps_cp.wait()

        lane = lax.iota(jnp.int32, LANES)
        ninf = jnp.full((LANES,), -jnp.inf, jnp.float32)
        sent = jnp.full((LANES,), N_TOKENS, jnp.int32)

        ps_vec = ps_v[...]
        vals_cp.wait()
        for k in range(B_SEGS):
            start = ps_vec[k - 1] if k > 0 else jnp.int32(0)
            end = ps_vec[k]
            a = jnp.clip(start - lo, 0, CHUNK)
            b = jnp.clip(end - lo, 0, CHUNK)
            ja = (a >> 4) << 4
            jb = ((b + (LANES - 1)) >> 4) << 4

            @pl.loop(ja, jb, init_carry=(ninf, sent), step=UNROLL * LANES)
            def scan(j, carry, a=a, b=b):
                cm, ci = carry
                for u in range(UNROLL):
                    v = vals_v[pl.ds(j + u * LANES, LANES)]
                    pos = j + (lane + u * LANES)
                    valid = (pos >= a) & (pos < b)
                    vm = jnp.where(valid, v, ninf)
                    upd = vm > cm
                    cm = jnp.where(upd, vm, cm)
                    ci = jnp.where(upd, pos, ci)
                return cm, ci

            cm, ci = scan
            cmtab_v[pl.ds(k * LANES, LANES)] = cm
            citab_v[pl.ds(k * LANES, LANES)] = ci + lo

        cols = []
        for l in range(LANES):
            tm = plsc.load_gather(cmtab_v, [lane * LANES + l])
            ti = plsc.load_gather(citab_v, [lane * LANES + l])
            cols.append((tm, ti))
        pm, pi = _tree_merge(cols)

        pub_v[pl.ds(0, LANES)] = pm
        pub_v[pl.ds(LANES, LANES)] = plsc.bitcast(pi, jnp.float32)
        pltpu.sync_copy(pub_v, sh_pub.at[pl.ds(sid * 2 * LANES, 2 * LANES)])
        plsc.subcore_barrier()

        @pl.when(sid == 0)
        def _():
            pltpu.sync_copy(sh_pub, mrg_v)
            rows = []
            for w in range(N_WORKERS):
                wm = mrg_v[pl.ds(w * 2 * LANES, LANES)]
                wi = plsc.bitcast(mrg_v[pl.ds(w * 2 * LANES + LANES, LANES)],
                                  jnp.int32)
                rows.append((wm, wi))
            cm, ci = _tree_merge(rows)
            ans_v[...] = jnp.where(ci >= N_TOKENS, -1, ci)
            pltpu.sync_copy(ans_v, out_hbm)


@jax.jit
def kernel(values, prefix_sum):
    ps32 = prefix_sum.astype(jnp.int32)
    mesh = plsc.VectorSubcoreMesh(
        core_axis_name="c", subcore_axis_name="s", num_cores=1, num_subcores=16
    )
    out = pl.kernel(
        _jagged_argmax_body,
        out_type=jax.ShapeDtypeStruct((B_SEGS,), jnp.int32),
        mesh=mesh,
        compiler_params=pltpu.CompilerParams(needs_layout_passes=False),
        scratch_types=[
            pltpu.VMEM((CHUNK + PAD,), jnp.float32),
            pltpu.VMEM((B_SEGS,), jnp.int32),
            pltpu.VMEM((2 * LANES,), jnp.float32),
            pltpu.VMEM((N_WORKERS * 2 * LANES,), jnp.float32),
            pltpu.VMEM((B_SEGS,), jnp.int32),
            pltpu.VMEM((B_SEGS * LANES,), jnp.float32),
            pltpu.VMEM((B_SEGS * LANES,), jnp.int32),
            pltpu.VMEM_SHARED((N_WORKERS * 2 * LANES,), jnp.float32),
            pltpu.SemaphoreType.DMA,
            pltpu.SemaphoreType.DMA,
        ],
    )(values, ps32)
    return out.astype(prefix_sum.dtype)


# speedup vs baseline: 1.0217x; 1.0064x over previous
import jax
import jax.numpy as jnp
from jax import lax
from jax.experimental import pallas as pl
from jax.experimental.pallas import tpu as pltpu
from jax.experimental.pallas import tpu_sc as plsc

N_TOKENS = 32768
B_SEGS = 16
N_WORKERS = 16
CHUNK = N_TOKENS // N_WORKERS
LANES = 16
UNROLL = 1
PAD = UNROLL * LANES


def _merge(am, ai, bm, bi):
    better = (bm > am) | ((bm == am) & (bi < ai))
    return jnp.where(better, bm, am), jnp.where(better, bi, ai)


def _tree_merge(pairs):
    while len(pairs) > 1:
        nxt = []
        for i in range(0, len(pairs) - 1, 2):
            am, ai = pairs[i]
            bm, bi = pairs[i + 1]
            nxt.append(_merge(am, ai, bm, bi))
        if len(pairs) % 2:
            nxt.append(pairs[-1])
        pairs = nxt
    return pairs[0]


def _jagged_argmax_body(values_hbm, ps_hbm, out_hbm,
                        vals_v, ps_v, pub_v, mrg_v, ans_v,
                        cmtab_v, citab_v, sh_pub, vals_sem, ps_sem):
    cid = lax.axis_index("c")
    sid = lax.axis_index("s")

    @pl.when(cid == 0)
    def _():
        lo = sid * CHUNK
        vals_cp = pltpu.async_copy(values_hbm.at[pl.ds(lo, CHUNK)],
                                   vals_v.at[pl.ds(0, CHUNK)], vals_sem)
        ps_cp = pltpu.async_copy(ps_hbm, ps_v, ps_sem)
        ps_cp.wait()

        lane = lax.iota(jnp.int32, LANES)
        ninf = jnp.full((LANES,), -jnp.inf, jnp.float32)
        sent = jnp.full((LANES,), N_TOKENS, jnp.int32)

        ps_vec = ps_v[...]
        vals_cp.wait()
        for k in range(B_SEGS):
            start = ps_vec[k - 1] if k > 0 else jnp.int32(0)
            end = ps_vec[k]
            a = jnp.maximum(start, lo)
            b = jnp.minimum(end, lo + CHUNK)
            va = lax.shift_right_logical(a, 4)
            vb = lax.shift_right_logical(b + (LANES - 1), 4)

            @pl.loop(va, vb, init_carry=(ninf, sent), step=UNROLL)
            def scan(j, carry, a=a, b=b, lo=lo):
                cm, ci = carry
                for u in range(UNROLL):
                    off = (j + u) * LANES
                    v = vals_v[pl.ds(off - lo, LANES)]
                    pos = off + lane
                    valid = (pos >= a) & (pos < b)
                    vm = jnp.where(valid, v, ninf)
                    upd = vm > cm
                    cm = jnp.where(upd, vm, cm)
                    ci = jnp.where(upd, pos, ci)
                return cm, ci

            cm, ci = scan
            cmtab_v[pl.ds(k * LANES, LANES)] = cm
            citab_v[pl.ds(k * LANES, LANES)] = ci

        cols = []
        for l in range(LANES):
            tm = plsc.load_gather(cmtab_v, [lane * LANES + l])
            ti = plsc.load_gather(citab_v, [lane * LANES + l])
            cols.append((tm, ti))
        pm, pi = _tree_merge(cols)

        pub_v[pl.ds(0, LANES)] = pm
        pub_v[pl.ds(LANES, LANES)] = plsc.bitcast(pi, jnp.float32)
        pltpu.sync_copy(pub_v, sh_pub.at[pl.ds(sid * 2 * LANES, 2 * LANES)])
        plsc.subcore_barrier()

        @pl.when(sid == 0)
        def _():
            pltpu.sync_copy(sh_pub, mrg_v)
            rows = []
            for w in range(N_WORKERS):
                wm = mrg_v[pl.ds(w * 2 * LANES, LANES)]
                wi = plsc.bitcast(mrg_v[pl.ds(w * 2 * LANES + LANES, LANES)],
                                  jnp.int32)
                rows.append((wm, wi))
            cm, ci = _tree_merge(rows)
            ans_v[...] = jnp.where(ci >= N_TOKENS, -1, ci)
            pltpu.sync_copy(ans_v, out_hbm)


@jax.jit
def kernel(values, prefix_sum):
    ps32 = prefix_sum.astype(jnp.int32)
    mesh = plsc.VectorSubcoreMesh(
        core_axis_name="c", subcore_axis_name="s", num_cores=1, num_subcores=16
    )
    out = pl.kernel(
        _jagged_argmax_body,
        out_type=jax.ShapeDtypeStruct((B_SEGS,), jnp.int32),
        mesh=mesh,
        compiler_params=pltpu.CompilerParams(needs_layout_passes=False),
        scratch_types=[
            pltpu.VMEM((CHUNK + PAD,), jnp.float32),
            pltpu.VMEM((B_SEGS,), jnp.int32),
            pltpu.VMEM((2 * LANES,), jnp.float32),
            pltpu.VMEM((N_WORKERS * 2 * LANES,), jnp.float32),
            pltpu.VMEM((B_SEGS,), jnp.int32),
            pltpu.VMEM((B_SEGS * LANES,), jnp.float32),
            pltpu.VMEM((B_SEGS * LANES,), jnp.int32),
            pltpu.VMEM_SHARED((N_WORKERS * 2 * LANES,), jnp.float32),
            pltpu.SemaphoreType.DMA,
            pltpu.SemaphoreType.DMA,
        ],
    )(values, ps32)
    return out.astype(prefix_sum.dtype)


# speedup vs baseline: 1.0691x; 1.0463x over previous
import jax
import jax.numpy as jnp
from jax import lax
from jax.experimental import pallas as pl
from jax.experimental.pallas import tpu as pltpu
from jax.experimental.pallas import tpu_sc as plsc

N_TOKENS = 32768
B_SEGS = 16
N_WORKERS = 16
CHUNK = N_TOKENS // N_WORKERS
LANES = 16
UNROLL = 2
PAD = UNROLL * LANES


def _merge(am, ai, bm, bi):
    better = (bm > am) | ((bm == am) & (bi < ai))
    return jnp.where(better, bm, am), jnp.where(better, bi, ai)


def _tree_merge(pairs):
    while len(pairs) > 1:
        nxt = []
        for i in range(0, len(pairs) - 1, 2):
            am, ai = pairs[i]
            bm, bi = pairs[i + 1]
            nxt.append(_merge(am, ai, bm, bi))
        if len(pairs) % 2:
            nxt.append(pairs[-1])
        pairs = nxt
    return pairs[0]


def _jagged_argmax_body(values_hbm, ps_hbm, out_hbm,
                        vals_v, ps_v, pub_v, mrg_v, ans_v,
                        cmtab_v, citab_v, sh_pub, vals_sem, ps_sem):
    cid = lax.axis_index("c")
    sid = lax.axis_index("s")

    @pl.when(cid == 0)
    def _():
        lo = sid * CHUNK
        vals_cp = pltpu.async_copy(values_hbm.at[pl.ds(lo, CHUNK)],
                                   vals_v.at[pl.ds(0, CHUNK)], vals_sem)
        ps_cp = pltpu.async_copy(ps_hbm, ps_v, ps_sem)
        ps_cp.wait()

        lane = lax.iota(jnp.int32, LANES)
        ninf = jnp.full((LANES,), -jnp.inf, jnp.float32)
        sent = jnp.full((LANES,), N_TOKENS, jnp.int32)

        ps_vec = ps_v[...]
        vals_cp.wait()
        for k in range(B_SEGS):
            start = ps_vec[k - 1] if k > 0 else jnp.int32(0)
            end = ps_vec[k]
            a = jnp.maximum(start, lo)
            b = jnp.minimum(end, lo + CHUNK)
            va = lax.shift_right_logical(a, 4)
            vb = lax.shift_right_logical(b + (LANES - 1), 4)

            @pl.loop(va, vb, init_carry=(ninf, sent), step=UNROLL)
            def scan(j, carry, a=a, b=b, lo=lo):
                cm, ci = carry
                for u in range(UNROLL):
                    off = (j + u) * LANES
                    v = vals_v[pl.ds(off - lo, LANES)]
                    pos = off + lane
                    valid = (pos >= a) & (pos < b)
                    vm = jnp.where(valid, v, ninf)
                    upd = vm > cm
                    cm = jnp.where(upd, vm, cm)
                    ci = jnp.where(upd, pos, ci)
                return cm, ci

            cm, ci = scan
            cmtab_v[pl.ds(k * LANES, LANES)] = cm
            citab_v[pl.ds(k * LANES, LANES)] = ci

        cols = []
        for l in range(LANES):
            tm = plsc.load_gather(cmtab_v, [lane * LANES + l])
            ti = plsc.load_gather(citab_v, [lane * LANES + l])
            cols.append((tm, ti))
        pm, pi = _tree_merge(cols)

        pub_v[pl.ds(0, LANES)] = pm
        pub_v[pl.ds(LANES, LANES)] = plsc.bitcast(pi, jnp.float32)
        pltpu.sync_copy(pub_v, sh_pub.at[pl.ds(sid * 2 * LANES, 2 * LANES)])
        plsc.subcore_barrier()

        @pl.when(sid == 0)
        def _():
            pltpu.sync_copy(sh_pub, mrg_v)
            rows = []
            for w in range(N_WORKERS):
                wm = mrg_v[pl.ds(w * 2 * LANES, LANES)]
                wi = plsc.bitcast(mrg_v[pl.ds(w * 2 * LANES + LANES, LANES)],
                                  jnp.int32)
                rows.append((wm, wi))
            cm, ci = _tree_merge(rows)
            ans_v[...] = jnp.where(ci >= N_TOKENS, -1, ci)
            pltpu.sync_copy(ans_v, out_hbm)


@jax.jit
def kernel(values, prefix_sum):
    ps32 = prefix_sum.astype(jnp.int32)
    mesh = plsc.VectorSubcoreMesh(
        core_axis_name="c", subcore_axis_name="s", num_cores=1, num_subcores=16
    )
    out = pl.kernel(
        _jagged_argmax_body,
        out_type=jax.ShapeDtypeStruct((B_SEGS,), jnp.int32),
        mesh=mesh,
        compiler_params=pltpu.CompilerParams(needs_layout_passes=False),
        scratch_types=[
            pltpu.VMEM((CHUNK + PAD,), jnp.float32),
            pltpu.VMEM((B_SEGS,), jnp.int32),
            pltpu.VMEM((2 * LANES,), jnp.float32),
            pltpu.VMEM((N_WORKERS * 2 * LANES,), jnp.float32),
            pltpu.VMEM((B_SEGS,), jnp.int32),
            pltpu.VMEM((B_SEGS * LANES,), jnp.float32),
            pltpu.VMEM((B_SEGS * LANES,), jnp.int32),
            pltpu.VMEM_SHARED((N_WORKERS * 2 * LANES,), jnp.float32),
            pltpu.SemaphoreType.DMA,
            pltpu.SemaphoreType.DMA,
        ],
    )(values, ps32)
    return out.astype(prefix_sum.dtype)
